# baseline (device time: 27230 ns/iter reference)
import jax
import jax.numpy as jnp
from jax import lax
from jax.experimental import pallas as pl
from jax.experimental.pallas import tpu as pltpu

B = 4
S = 512
S_OUT = 256
SQ = 128
K = 512
N = 1024
CPB = 2
CR = SQ // CPB
NC = B * CPB


def kernel(O, Wo):

    def body(o_ref, w_ref, out_ref, w_bf, xsend_buf, xrecv_buf,
             ysend_buf, yrecv_buf,
             xsend_sems, xrecv_sems, ysend_sems, yrecv_sems):
        my_x = lax.axis_index("x")
        my_y = lax.axis_index("y")
        ox = 1 - my_x
        oy = 1 - my_y

        barrier = pltpu.get_barrier_semaphore()
        pl.semaphore_signal(
            barrier, inc=1,
            device_id=(ox, my_y), device_id_type=pl.DeviceIdType.MESH,
        )
        pl.semaphore_signal(
            barrier, inc=1,
            device_id=(my_x, oy), device_id_type=pl.DeviceIdType.MESH,
        )

        w_bf[...] = w_ref[...].astype(jnp.bfloat16)

        my_q = my_x * S_OUT + my_y * SQ
        nb_q = ox * S_OUT + my_y * SQ
        loc = my_y * SQ

        x_rdmas = []
        for c in range(NC):
            b, half = divmod(c, CPB)
            xsend_buf[c] = jnp.dot(
                o_ref[b, pl.ds(nb_q + half * CR, CR), :, :]
                .astype(jnp.bfloat16).reshape(CR, K),
                w_bf[...],
                preferred_element_type=jnp.float32,
            ).astype(jnp.bfloat16)
            if c == 0:
                pl.semaphore_wait(barrier, 2)
            rdma = pltpu.make_async_remote_copy(
                src_ref=xsend_buf.at[c],
                dst_ref=xrecv_buf.at[c],
                send_sem=xsend_sems.at[c],
                recv_sem=xrecv_sems.at[c],
                device_id=(ox, my_y),
                device_id_type=pl.DeviceIdType.MESH,
            )
            rdma.start()
            x_rdmas.append(rdma)

        for b in range(B):
            out_ref[b, pl.ds(loc, SQ), :] = jnp.dot(
                o_ref[b, pl.ds(my_q, SQ), :, :]
                .astype(jnp.bfloat16).reshape(SQ, K),
                w_bf[...],
                preferred_element_type=jnp.float32,
            )

        y_rdmas = []
        for c in range(NC):
            b, half = divmod(c, CPB)
            row = loc + half * CR
            x_rdmas[c].wait()
            r = out_ref[b, pl.ds(row, CR), :] + xrecv_buf[c].astype(jnp.float32)
            out_ref[b, pl.ds(row, CR), :] = r
            ysend_buf[c] = r.astype(jnp.bfloat16)
            yr = pltpu.make_async_remote_copy(
                src_ref=ysend_buf.at[c],
                dst_ref=yrecv_buf.at[c],
                send_sem=ysend_sems.at[c],
                recv_sem=yrecv_sems.at[c],
                device_id=(my_x, oy),
                device_id_type=pl.DeviceIdType.MESH,
            )
            yr.start()
            y_rdmas.append(yr)

        for c in range(NC):
            b, half = divmod(c, CPB)
            row = oy * SQ + half * CR
            y_rdmas[c].wait()
            out_ref[b, pl.ds(row, CR), :] = yrecv_buf[c].astype(jnp.float32)

    return pl.pallas_call(
        body,
        out_shape=jax.ShapeDtypeStruct((B, S_OUT, N), jnp.float32),
        in_specs=[
            pl.BlockSpec(memory_space=pltpu.VMEM),
            pl.BlockSpec(memory_space=pltpu.VMEM),
        ],
        out_specs=pl.BlockSpec(memory_space=pltpu.VMEM),
        scratch_shapes=[
            pltpu.VMEM((K, N), jnp.bfloat16),
            pltpu.VMEM((NC, CR, N), jnp.bfloat16),
            pltpu.VMEM((NC, CR, N), jnp.bfloat16),
            pltpu.VMEM((NC, CR, N), jnp.bfloat16),
            pltpu.VMEM((NC, CR, N), jnp.bfloat16),
            pltpu.SemaphoreType.DMA((NC,)),
            pltpu.SemaphoreType.DMA((NC,)),
            pltpu.SemaphoreType.DMA((NC,)),
            pltpu.SemaphoreType.DMA((NC,)),
        ],
        compiler_params=pltpu.CompilerParams(collective_id=0),
    )(O, Wo)


# device time: 24689 ns/iter; 1.1029x vs baseline; 1.1029x over previous
import jax
import jax.numpy as jnp
from jax import lax
from jax.experimental import pallas as pl
from jax.experimental.pallas import tpu as pltpu

B = 4
S = 512
S_OUT = 256
SQ = 128
K = 512
N = 1024
CPB = 2
CR = SQ // CPB
NC = B * CPB


def kernel(O, Wo):
    my_x = lax.axis_index("x")
    my_y = lax.axis_index("y")
    my_q = my_x * S_OUT + my_y * SQ
    nb_q = (1 - my_x) * S_OUT + my_y * SQ
    O_nb = lax.dynamic_slice(O, (0, nb_q, 0, 0), (B, SQ, 8, 64)).reshape(B, SQ, K)
    O_my = lax.dynamic_slice(O, (0, my_q, 0, 0), (B, SQ, 8, 64)).reshape(B, SQ, K)

    def body(o_nb_ref, o_my_ref, w_ref, out_ref, w_bf, xsend_buf, xrecv_buf,
             ysend_buf, yrecv_buf,
             xsend_sems, xrecv_sems, ysend_sems, yrecv_sems):
        my_x = lax.axis_index("x")
        my_y = lax.axis_index("y")
        ox = 1 - my_x
        oy = 1 - my_y

        barrier = pltpu.get_barrier_semaphore()
        pl.semaphore_signal(
            barrier, inc=1,
            device_id=(ox, my_y), device_id_type=pl.DeviceIdType.MESH,
        )
        pl.semaphore_signal(
            barrier, inc=1,
            device_id=(my_x, oy), device_id_type=pl.DeviceIdType.MESH,
        )

        w_bf[...] = w_ref[...].astype(jnp.bfloat16)

        loc = my_y * SQ

        x_rdmas = []
        for c in range(NC):
            b, half = divmod(c, CPB)
            xsend_buf[c] = jnp.dot(
                o_nb_ref[b, pl.ds(half * CR, CR), :].astype(jnp.bfloat16),
                w_bf[...],
                preferred_element_type=jnp.float32,
            ).astype(jnp.bfloat16)
            if c == 0:
                pl.semaphore_wait(barrier, 2)
            rdma = pltpu.make_async_remote_copy(
                src_ref=xsend_buf.at[c],
                dst_ref=xrecv_buf.at[c],
                send_sem=xsend_sems.at[c],
                recv_sem=xrecv_sems.at[c],
                device_id=(ox, my_y),
                device_id_type=pl.DeviceIdType.MESH,
            )
            rdma.start()
            x_rdmas.append(rdma)

        for b in range(B):
            out_ref[b, pl.ds(loc, SQ), :] = jnp.dot(
                o_my_ref[b].astype(jnp.bfloat16),
                w_bf[...],
                preferred_element_type=jnp.float32,
            )

        y_rdmas = []
        for c in range(NC):
            b, half = divmod(c, CPB)
            row = loc + half * CR
            x_rdmas[c].wait()
            r = out_ref[b, pl.ds(row, CR), :] + xrecv_buf[c].astype(jnp.float32)
            out_ref[b, pl.ds(row, CR), :] = r
            ysend_buf[c] = r.astype(jnp.bfloat16)
            yr = pltpu.make_async_remote_copy(
                src_ref=ysend_buf.at[c],
                dst_ref=yrecv_buf.at[c],
                send_sem=ysend_sems.at[c],
                recv_sem=yrecv_sems.at[c],
                device_id=(my_x, oy),
                device_id_type=pl.DeviceIdType.MESH,
            )
            yr.start()
            y_rdmas.append(yr)

        for c in range(NC):
            b, half = divmod(c, CPB)
            row = oy * SQ + half * CR
            y_rdmas[c].wait()
            out_ref[b, pl.ds(row, CR), :] = yrecv_buf[c].astype(jnp.float32)

    return pl.pallas_call(
        body,
        out_shape=jax.ShapeDtypeStruct((B, S_OUT, N), jnp.float32),
        in_specs=[
            pl.BlockSpec(memory_space=pltpu.VMEM),
            pl.BlockSpec(memory_space=pltpu.VMEM),
            pl.BlockSpec(memory_space=pltpu.VMEM),
        ],
        out_specs=pl.BlockSpec(memory_space=pltpu.VMEM),
        scratch_shapes=[
            pltpu.VMEM((K, N), jnp.bfloat16),
            pltpu.VMEM((NC, CR, N), jnp.bfloat16),
            pltpu.VMEM((NC, CR, N), jnp.bfloat16),
            pltpu.VMEM((NC, CR, N), jnp.bfloat16),
            pltpu.VMEM((NC, CR, N), jnp.bfloat16),
            pltpu.SemaphoreType.DMA((NC,)),
            pltpu.SemaphoreType.DMA((NC,)),
            pltpu.SemaphoreType.DMA((NC,)),
            pltpu.SemaphoreType.DMA((NC,)),
        ],
        compiler_params=pltpu.CompilerParams(collective_id=0),
    )(O_nb, O_my, Wo)


# device time: 24101 ns/iter; 1.1298x vs baseline; 1.0244x over previous
import jax
import jax.numpy as jnp
from jax import lax
from jax.experimental import pallas as pl
from jax.experimental.pallas import tpu as pltpu

B = 4
S = 512
S_OUT = 256
SQ = 128
K = 512
N = 1024
CPB = 2
CR = SQ // CPB
NC = B * CPB


def kernel(O, Wo):
    my_x = lax.axis_index("x")
    my_y = lax.axis_index("y")
    my_q = my_x * S_OUT + my_y * SQ
    nb_q = (1 - my_x) * S_OUT + my_y * SQ
    O_nb = lax.dynamic_slice(O, (0, nb_q, 0, 0), (B, SQ, 8, 64)) \
        .reshape(B, SQ, K).astype(jnp.bfloat16)
    O_my = lax.dynamic_slice(O, (0, my_q, 0, 0), (B, SQ, 8, 64)) \
        .reshape(B, SQ, K).astype(jnp.bfloat16)
    Wo_bf = Wo.astype(jnp.bfloat16)

    def body(o_nb_ref, o_my_ref, w_ref, out_ref, xsend_buf, xrecv_buf,
             ysend_buf, yrecv_buf,
             xsend_sems, xrecv_sems, ysend_sems, yrecv_sems):
        my_x = lax.axis_index("x")
        my_y = lax.axis_index("y")
        ox = 1 - my_x
        oy = 1 - my_y

        barrier = pltpu.get_barrier_semaphore()
        pl.semaphore_signal(
            barrier, inc=1,
            device_id=(ox, my_y), device_id_type=pl.DeviceIdType.MESH,
        )
        pl.semaphore_signal(
            barrier, inc=1,
            device_id=(my_x, oy), device_id_type=pl.DeviceIdType.MESH,
        )

        loc = my_y * SQ

        x_rdmas = []
        for c in range(NC):
            b, half = divmod(c, CPB)
            xsend_buf[c] = jnp.dot(
                o_nb_ref[b, pl.ds(half * CR, CR), :],
                w_ref[...],
                preferred_element_type=jnp.float32,
            ).astype(jnp.bfloat16)
            if c == 0:
                pl.semaphore_wait(barrier, 2)
            rdma = pltpu.make_async_remote_copy(
                src_ref=xsend_buf.at[c],
                dst_ref=xrecv_buf.at[c],
                send_sem=xsend_sems.at[c],
                recv_sem=xrecv_sems.at[c],
                device_id=(ox, my_y),
                device_id_type=pl.DeviceIdType.MESH,
            )
            rdma.start()
            x_rdmas.append(rdma)

        for b in range(B):
            out_ref[b, pl.ds(loc, SQ), :] = jnp.dot(
                o_my_ref[b],
                w_ref[...],
                preferred_element_type=jnp.float32,
            )

        y_rdmas = []
        for c in range(NC):
            b, half = divmod(c, CPB)
            row = loc + half * CR
            x_rdmas[c].wait()
            r = out_ref[b, pl.ds(row, CR), :] + xrecv_buf[c].astype(jnp.float32)
            out_ref[b, pl.ds(row, CR), :] = r
            ysend_buf[c] = r.astype(jnp.bfloat16)
            yr = pltpu.make_async_remote_copy(
                src_ref=ysend_buf.at[c],
                dst_ref=yrecv_buf.at[c],
                send_sem=ysend_sems.at[c],
                recv_sem=yrecv_sems.at[c],
                device_id=(my_x, oy),
                device_id_type=pl.DeviceIdType.MESH,
            )
            yr.start()
            y_rdmas.append(yr)

        for c in range(NC):
            b, half = divmod(c, CPB)
            row = oy * SQ + half * CR
            y_rdmas[c].wait()
            out_ref[b, pl.ds(row, CR), :] = yrecv_buf[c].astype(jnp.float32)

    return pl.pallas_call(
        body,
        out_shape=jax.ShapeDtypeStruct((B, S_OUT, N), jnp.float32),
        in_specs=[
            pl.BlockSpec(memory_space=pltpu.VMEM),
            pl.BlockSpec(memory_space=pltpu.VMEM),
            pl.BlockSpec(memory_space=pltpu.VMEM),
        ],
        out_specs=pl.BlockSpec(memory_space=pltpu.VMEM),
        scratch_shapes=[
            pltpu.VMEM((NC, CR, N), jnp.bfloat16),
            pltpu.VMEM((NC, CR, N), jnp.bfloat16),
            pltpu.VMEM((NC, CR, N), jnp.bfloat16),
            pltpu.VMEM((NC, CR, N), jnp.bfloat16),
            pltpu.SemaphoreType.DMA((NC,)),
            pltpu.SemaphoreType.DMA((NC,)),
            pltpu.SemaphoreType.DMA((NC,)),
            pltpu.SemaphoreType.DMA((NC,)),
        ],
        compiler_params=pltpu.CompilerParams(collective_id=0),
    )(O_nb, O_my, Wo_bf)
